# baseline (speedup 1.0000x reference)
import functools

import jax
import jax.numpy as jnp
from jax import lax
from jax.experimental import pallas as pl
from jax.experimental.pallas import tpu as pltpu
from jax.experimental.pallas import tpu_sc as plsc

N = 10000
E = 320000
D = 128
NC = 2
NS = 16
NW = NC * NS
EP_TILE = 10080
EP = EP_TILE * NW
KW = 112
NWIN = EP_TILE // KW
NBUF = 3
NACC2 = 10112
NROW_T2 = NACC2 // NS
NP = 10240
NACC = NP
ZCH = NACC // NS
NROW_T = NP // NS
BN = 1024

def _mesh():
    return plsc.VectorSubcoreMesh(
        core_axis_name="c", subcore_axis_name="s",
        num_cores=NC, num_subcores=NS)


KW1 = 112
NWIN1 = EP_TILE // KW1
NB1 = 3


def _rowsum_body(row_hbm, ev_hbm, z_hbm, out_hbm, *sc):
    ebufs = sc[0:NB1]
    evfs = sc[NB1:2 * NB1]
    acc = sc[2 * NB1]
    sem = sc[2 * NB1 + 1]
    esems = sc[2 * NB1 + 2:2 * NB1 + 2 + NB1]
    ssems = sc[2 * NB1 + 2 + NB1:2 * NB1 + 2 + 2 * NB1]

    c = lax.axis_index("c")
    s = lax.axis_index("s")
    wid = c * NS + s
    base = wid * EP_TILE

    zcp = pltpu.async_copy(z_hbm.at[pl.ds(s * ZCH, ZCH)],
                           acc.at[pl.ds(s * ZCH, ZCH)], sem)

    def estart(b, w):
        pltpu.async_copy(row_hbm.at[pl.ds(base + w * KW1, KW1)],
                         ebufs[b], esems[b])
        pltpu.async_copy(ev_hbm.at[pl.ds(base + w * KW1, KW1)],
                         evfs[b], esems[b])

    def ewait(b, w):
        pltpu.make_async_copy(row_hbm.at[pl.ds(base + w * KW1, KW1)],
                              ebufs[b], esems[b]).wait()
        pltpu.make_async_copy(ev_hbm.at[pl.ds(base + w * KW1, KW1)],
                              evfs[b], esems[b]).wait()

    def sstart(b):
        pltpu.async_copy(evfs[b], acc.at[ebufs[b]], ssems[b], add=True)

    def swait(b):
        pltpu.make_async_copy(evfs[b], acc.at[ebufs[b]], ssems[b]).wait()

    def win_ops(w, b):
        @pl.when(w >= 1)
        def _():
            swait((b + 2) % NB1)

        @pl.when(w <= NWIN1 - 3)
        def _():
            estart((b + 2) % NB1, w + 2)

        ewait(b, w)
        sstart(b)

    estart(0, 0)
    estart(1, 1)
    zcp.wait()
    plsc.subcore_barrier()
    lax.fori_loop(
        0, NWIN1 // NB1,
        lambda k, carry: ([win_ops(NB1 * k + j, j) for j in range(NB1)],
                          carry)[1], 0)
    swait((NWIN1 - 1) % NB1)
    plsc.subcore_barrier()
    pltpu.sync_copy(acc.at[pl.ds(s * ZCH, ZCH)],
                    out_hbm.at[c, pl.ds(s * ZCH, ZCH)])


@functools.cache
def _rowsum_call():
    return pl.kernel(
        _rowsum_body,
        out_type=jax.ShapeDtypeStruct((NC, NACC), jnp.float32),
        mesh=_mesh(),
        scratch_types=(
            [pltpu.VMEM((KW1,), jnp.int32) for _ in range(NB1)]
            + [pltpu.VMEM((KW1,), jnp.float32) for _ in range(NB1)]
            + [pltpu.VMEM_SHARED((NACC,), jnp.float32)]
            + [pltpu.SemaphoreType.DMA] * (1 + 2 * NB1)
        ),
    )


def _spmm_body(g_hbm, row_hbm, col_hbm, ev_hbm, z_hbm, out_hbm, *sc):
    rowbufs = sc[0:NBUF]
    colbufs = sc[NBUF:2 * NBUF]
    evbufs = sc[2 * NBUF:3 * NBUF]
    rows = sc[3 * NBUF:4 * NBUF]
    acc = sc[4 * NBUF]
    sem = sc[4 * NBUF + 1]
    esems = sc[4 * NBUF + 2:4 * NBUF + 2 + NBUF]
    gsems = sc[4 * NBUF + 2 + NBUF:4 * NBUF + 2 + 2 * NBUF]
    ssems = sc[4 * NBUF + 2 + 2 * NBUF:4 * NBUF + 2 + 3 * NBUF]

    c = lax.axis_index("c")
    s = lax.axis_index("s")
    wid = c * NS + s
    base = wid * EP_TILE

    zcp = pltpu.async_copy(z_hbm.at[pl.ds(s * NROW_T2, NROW_T2)],
                           acc.at[pl.ds(s * NROW_T2, NROW_T2)], sem)

    def estart(b, w):
        o = base + w * KW
        pltpu.async_copy(row_hbm.at[pl.ds(o, KW)], rowbufs[b], esems[b])
        pltpu.async_copy(col_hbm.at[pl.ds(o, KW)], colbufs[b], esems[b])
        pltpu.async_copy(ev_hbm.at[pl.ds(o, KW)], evbufs[b], esems[b])

    def ewait(b, w):
        o = base + w * KW
        pltpu.make_async_copy(
            row_hbm.at[pl.ds(o, KW)], rowbufs[b], esems[b]).wait()
        pltpu.make_async_copy(
            col_hbm.at[pl.ds(o, KW)], colbufs[b], esems[b]).wait()
        pltpu.make_async_copy(
            ev_hbm.at[pl.ds(o, KW)], evbufs[b], esems[b]).wait()

    def gstart(b):
        pltpu.async_copy(g_hbm.at[colbufs[b]], rows[b], gsems[b])

    def gwait(b):
        pltpu.make_async_copy(g_hbm.at[colbufs[b]], rows[b], gsems[b]).wait()

    def scale(b):
        def scale16(e16, carry2):
            e0 = e16 * 16
            ew16 = evbufs[b][pl.ds(e0, 16)]
            for j in range(16):
                bc = jnp.full((16,), ew16[j], jnp.float32)
                for f in range(D // 16):
                    rows[b][e0 + j, pl.ds(f * 16, 16)] = (
                        rows[b][e0 + j, pl.ds(f * 16, 16)] * bc)
            return carry2

        lax.fori_loop(0, KW // 16, scale16, 0)

    def sstart(b):
        pltpu.async_copy(rows[b], acc.at[rowbufs[b]], ssems[b], add=True)

    def swait(b):
        pltpu.make_async_copy(rows[b], acc.at[rowbufs[b]], ssems[b]).wait()

    def win_ops(w, b):
        bp = (b + 2) % NBUF
        bg = (b + 1) % NBUF

        @pl.when(w >= 1)
        def _():
            swait(bp)

        @pl.when(w <= NWIN - 3)
        def _():
            estart(bp, w + 2)

        @pl.when(w <= NWIN - 2)
        def _():
            ewait(bg, w + 1)
            gstart(bg)

        gwait(b)
        scale(b)
        sstart(b)

    estart(0, 0)
    estart(1, 1)
    ewait(0, 0)
    gstart(0)
    zcp.wait()
    plsc.subcore_barrier()
    lax.fori_loop(
        0, NWIN // NBUF,
        lambda k, carry: ([win_ops(NBUF * k + j, j) for j in range(NBUF)],
                          carry)[1], 0)
    swait((NWIN - 1) % NBUF)
    plsc.subcore_barrier()
    pltpu.sync_copy(acc.at[pl.ds(s * NROW_T2, NROW_T2)],
                    out_hbm.at[c, pl.ds(s * NROW_T2, NROW_T2)])


@functools.cache
def _spmm_call():
    return pl.kernel(
        _spmm_body,
        out_type=jax.ShapeDtypeStruct((NC, NP, D), jnp.float32),
        mesh=_mesh(),
        scratch_types=(
            [pltpu.VMEM((KW,), jnp.int32) for _ in range(NBUF)]
            + [pltpu.VMEM((KW,), jnp.int32) for _ in range(NBUF)]
            + [pltpu.VMEM((KW,), jnp.float32) for _ in range(NBUF)]
            + [pltpu.VMEM((KW, D), jnp.float32) for _ in range(NBUF)]
            + [pltpu.VMEM_SHARED((NACC2, D), jnp.float32)]
            + [pltpu.SemaphoreType.DMA] * (1 + 3 * NBUF)
        ),
    )


def _dvec(rsp_ref):
    rs = rsp_ref[0, :] + rsp_ref[1, :] + 1e-6
    return jnp.clip(lax.rsqrt(rs), 0.0, 10.0)


def _k2_body(rsp_ref, x_ref, w0_ref, g1_ref):
    dv = _dvec(rsp_ref)
    xw = jnp.dot(x_ref[...], w0_ref[...], preferred_element_type=jnp.float32)
    g1_ref[...] = dv[:, None] * xw


def _k5_body(rsp_ref, hp_ref, w1_ref, g2_ref):
    dv = _dvec(rsp_ref)
    h = jax.nn.relu(dv[:, None] * (hp_ref[0] + hp_ref[1]))
    hw = jnp.dot(h, w1_ref[...], preferred_element_type=jnp.float32)
    g2_ref[...] = dv[:, None] * hw


def _k6_body(rsp_ref, op_ref, out_ref):
    dv = _dvec(rsp_ref)
    out_ref[...] = dv[:, None] * (op_ref[0] + op_ref[1])


_rsp_spec = pl.BlockSpec((NC, BN), lambda i: (0, i))
_mat_spec = pl.BlockSpec((BN, D), lambda i: (i, 0))
_par_spec = pl.BlockSpec((NC, BN, D), lambda i: (0, i, 0))
_w_spec = pl.BlockSpec((D, D), lambda i: (0, 0))

_k2_call = pl.pallas_call(
    _k2_body,
    grid=(NP // BN,),
    in_specs=[_rsp_spec, _mat_spec, _w_spec],
    out_specs=_mat_spec,
    out_shape=jax.ShapeDtypeStruct((NP, D), jnp.float32),
)

_k5_call = pl.pallas_call(
    _k5_body,
    grid=(NP // BN,),
    in_specs=[_rsp_spec, _par_spec, _w_spec],
    out_specs=_mat_spec,
    out_shape=jax.ShapeDtypeStruct((NP, D), jnp.float32),
)

_k6_call = pl.pallas_call(
    _k6_body,
    grid=(NP // BN,),
    in_specs=[_rsp_spec, _par_spec],
    out_specs=_mat_spec,
    out_shape=jax.ShapeDtypeStruct((NP, D), jnp.float32),
)


def kernel(x, edge_index, edge_values, W0, W1):
    row = edge_index[0]
    col = edge_index[1]
    pad = EP - E
    pad_idx = (jnp.arange(pad, dtype=jnp.int32) % N)
    row_p = jnp.concatenate([row, pad_idx])
    col_p = jnp.concatenate([col, pad_idx])
    ev_p = jnp.concatenate([edge_values, jnp.zeros((pad,), jnp.float32)])
    z1 = jnp.zeros((NACC,), jnp.float32)
    z2 = jnp.zeros((NACC2, D), jnp.float32)
    x_p = jnp.concatenate([x, jnp.zeros((NP - N, D), jnp.float32)])

    rsp = _rowsum_call()(row_p, ev_p, z1)
    g1 = _k2_call(rsp, x_p, W0)
    hp = _spmm_call()(g1, row_p, col_p, ev_p, z2)
    g2 = _k5_call(rsp, hp, W1)
    op = _spmm_call()(g2, row_p, col_p, ev_p, z2)
    return _k6_call(rsp, op)[:N]


# speedup vs baseline: 1.0317x; 1.0317x over previous
import functools

import jax
import jax.numpy as jnp
from jax import lax
from jax.experimental import pallas as pl
from jax.experimental.pallas import tpu as pltpu
from jax.experimental.pallas import tpu_sc as plsc

N = 10000
E = 320000
D = 128
NC = 2
NS = 16
NW = NC * NS
EP_TILE = 10240
EP = EP_TILE * NW
KW = 80
NWIN = EP_TILE // KW
NBUF = 4
NP = 10240
NACC = NP
ZCH = NACC // NS
NROW_T = NP // NS
BN = 1024
NACC2 = NP
NROW_T2 = NACC2 // NS

def _mesh():
    return plsc.VectorSubcoreMesh(
        core_axis_name="c", subcore_axis_name="s",
        num_cores=NC, num_subcores=NS)


KW1 = 80
NWIN1 = EP_TILE // KW1
NB1 = 4


def _rowsum_body(row_hbm, ev_hbm, z_hbm, out_hbm, *sc):
    ebufs = sc[0:NB1]
    evfs = sc[NB1:2 * NB1]
    acc = sc[2 * NB1]
    sem = sc[2 * NB1 + 1]
    esems = sc[2 * NB1 + 2:2 * NB1 + 2 + NB1]
    ssems = sc[2 * NB1 + 2 + NB1:2 * NB1 + 2 + 2 * NB1]

    c = lax.axis_index("c")
    s = lax.axis_index("s")
    wid = c * NS + s
    base = wid * EP_TILE

    zcp = pltpu.async_copy(z_hbm.at[pl.ds(s * ZCH, ZCH)],
                           acc.at[pl.ds(s * ZCH, ZCH)], sem)

    def estart(b, w):
        pltpu.async_copy(row_hbm.at[pl.ds(base + w * KW1, KW1)],
                         ebufs[b], esems[b])
        pltpu.async_copy(ev_hbm.at[pl.ds(base + w * KW1, KW1)],
                         evfs[b], esems[b])

    def ewait(b, w):
        pltpu.make_async_copy(row_hbm.at[pl.ds(base + w * KW1, KW1)],
                              ebufs[b], esems[b]).wait()
        pltpu.make_async_copy(ev_hbm.at[pl.ds(base + w * KW1, KW1)],
                              evfs[b], esems[b]).wait()

    def sstart(b):
        pltpu.async_copy(evfs[b], acc.at[ebufs[b]], ssems[b], add=True)

    def swait(b):
        pltpu.make_async_copy(evfs[b], acc.at[ebufs[b]], ssems[b]).wait()

    def win_ops(w, b):
        @pl.when(w >= NB1 - 2)
        def _():
            swait((b + 2) % NB1)

        @pl.when(w <= NWIN1 - 3)
        def _():
            estart((b + 2) % NB1, w + 2)

        ewait(b, w)
        sstart(b)

    estart(0, 0)
    estart(1, 1)
    zcp.wait()
    plsc.subcore_barrier()
    lax.fori_loop(
        0, NWIN1 // NB1,
        lambda k, carry: ([win_ops(NB1 * k + j, j) for j in range(NB1)],
                          carry)[1], 0)
    for wl in range(NWIN1 - (NB1 - 2), NWIN1):
        swait(wl % NB1)
    plsc.subcore_barrier()
    pltpu.sync_copy(acc.at[pl.ds(s * ZCH, ZCH)],
                    out_hbm.at[c, pl.ds(s * ZCH, ZCH)])


@functools.cache
def _rowsum_call():
    return pl.kernel(
        _rowsum_body,
        out_type=jax.ShapeDtypeStruct((NC, NACC), jnp.float32),
        mesh=_mesh(),
        scratch_types=(
            [pltpu.VMEM((KW1,), jnp.int32) for _ in range(NB1)]
            + [pltpu.VMEM((KW1,), jnp.float32) for _ in range(NB1)]
            + [pltpu.VMEM_SHARED((NACC,), jnp.float32)]
            + [pltpu.SemaphoreType.DMA] * (1 + 2 * NB1)
        ),
    )


def _spmm_body(g_hbm, row_hbm, col_hbm, ev_hbm, z_hbm, out_hbm, *sc):
    rowbufs = sc[0:NBUF]
    colbufs = sc[NBUF:2 * NBUF]
    evbufs = sc[2 * NBUF:3 * NBUF]
    rows = sc[3 * NBUF:4 * NBUF]
    acc = sc[4 * NBUF]
    sem = sc[4 * NBUF + 1]
    esems = sc[4 * NBUF + 2:4 * NBUF + 2 + NBUF]
    gsems = sc[4 * NBUF + 2 + NBUF:4 * NBUF + 2 + 2 * NBUF]
    ssems = sc[4 * NBUF + 2 + 2 * NBUF:4 * NBUF + 2 + 3 * NBUF]

    c = lax.axis_index("c")
    s = lax.axis_index("s")
    wid = c * NS + s
    base = wid * EP_TILE

    zcp = pltpu.async_copy(z_hbm.at[pl.ds(s * NROW_T2, NROW_T2)],
                           acc.at[pl.ds(s * NROW_T2, NROW_T2)], sem)

    def estart(b, w):
        o = base + w * KW
        pltpu.async_copy(row_hbm.at[pl.ds(o, KW)], rowbufs[b], esems[b])
        pltpu.async_copy(col_hbm.at[pl.ds(o, KW)], colbufs[b], esems[b])
        pltpu.async_copy(ev_hbm.at[pl.ds(o, KW)], evbufs[b], esems[b])

    def ewait(b, w):
        o = base + w * KW
        pltpu.make_async_copy(
            row_hbm.at[pl.ds(o, KW)], rowbufs[b], esems[b]).wait()
        pltpu.make_async_copy(
            col_hbm.at[pl.ds(o, KW)], colbufs[b], esems[b]).wait()
        pltpu.make_async_copy(
            ev_hbm.at[pl.ds(o, KW)], evbufs[b], esems[b]).wait()

    def gstart(b):
        pltpu.async_copy(g_hbm.at[colbufs[b]], rows[b], gsems[b])

    def gwait(b):
        pltpu.make_async_copy(g_hbm.at[colbufs[b]], rows[b], gsems[b]).wait()

    def scale(b):
        def scale16(e16, carry2):
            e0 = e16 * 16
            ew16 = evbufs[b][pl.ds(e0, 16)]
            for j in range(16):
                bc = jnp.full((16,), ew16[j], jnp.float32)
                for f in range(D // 16):
                    rows[b][e0 + j, pl.ds(f * 16, 16)] = (
                        rows[b][e0 + j, pl.ds(f * 16, 16)] * bc)
            return carry2

        lax.fori_loop(0, KW // 16, scale16, 0)

    def sstart(b):
        pltpu.async_copy(rows[b], acc.at[rowbufs[b]], ssems[b], add=True)

    def swait(b):
        pltpu.make_async_copy(rows[b], acc.at[rowbufs[b]], ssems[b]).wait()

    def win_ops(w, b):
        bp = (b + 2) % NBUF
        bg = (b + 1) % NBUF

        @pl.when(w >= NBUF - 2)
        def _():
            swait(bp)

        @pl.when(w <= NWIN - 3)
        def _():
            estart(bp, w + 2)

        @pl.when(w <= NWIN - 2)
        def _():
            ewait(bg, w + 1)
            gstart(bg)

        gwait(b)
        scale(b)
        sstart(b)

    estart(0, 0)
    estart(1, 1)
    ewait(0, 0)
    gstart(0)
    zcp.wait()
    plsc.subcore_barrier()
    lax.fori_loop(
        0, NWIN // NBUF,
        lambda k, carry: ([win_ops(NBUF * k + j, j) for j in range(NBUF)],
                          carry)[1], 0)
    for wl in range(NWIN - (NBUF - 2), NWIN):
        swait(wl % NBUF)
    plsc.subcore_barrier()
    pltpu.sync_copy(acc.at[pl.ds(s * NROW_T2, NROW_T2)],
                    out_hbm.at[c, pl.ds(s * NROW_T2, NROW_T2)])


@functools.cache
def _spmm_call():
    return pl.kernel(
        _spmm_body,
        out_type=jax.ShapeDtypeStruct((NC, NP, D), jnp.float32),
        mesh=_mesh(),
        scratch_types=(
            [pltpu.VMEM((KW,), jnp.int32) for _ in range(NBUF)]
            + [pltpu.VMEM((KW,), jnp.int32) for _ in range(NBUF)]
            + [pltpu.VMEM((KW,), jnp.float32) for _ in range(NBUF)]
            + [pltpu.VMEM((KW, D), jnp.float32) for _ in range(NBUF)]
            + [pltpu.VMEM_SHARED((NACC2, D), jnp.float32)]
            + [pltpu.SemaphoreType.DMA] * (1 + 3 * NBUF)
        ),
    )


def _dvec(rsp_ref):
    rs = rsp_ref[0, :] + rsp_ref[1, :] + 1e-6
    return jnp.clip(lax.rsqrt(rs), 0.0, 10.0)


def _k2_body(rsp_ref, x_ref, w0_ref, g1_ref):
    dv = _dvec(rsp_ref)
    xw = jnp.dot(x_ref[...], w0_ref[...], preferred_element_type=jnp.float32)
    g1_ref[...] = dv[:, None] * xw


def _k5_body(rsp_ref, hp_ref, w1_ref, g2_ref):
    dv = _dvec(rsp_ref)
    h = jax.nn.relu(dv[:, None] * (hp_ref[0] + hp_ref[1]))
    hw = jnp.dot(h, w1_ref[...], preferred_element_type=jnp.float32)
    g2_ref[...] = dv[:, None] * hw


def _k6_body(rsp_ref, op_ref, out_ref):
    dv = _dvec(rsp_ref)
    out_ref[...] = dv[:, None] * (op_ref[0] + op_ref[1])


_rsp_spec = pl.BlockSpec((NC, BN), lambda i: (0, i))
_mat_spec = pl.BlockSpec((BN, D), lambda i: (i, 0))
_par_spec = pl.BlockSpec((NC, BN, D), lambda i: (0, i, 0))
_w_spec = pl.BlockSpec((D, D), lambda i: (0, 0))

_k2_call = pl.pallas_call(
    _k2_body,
    grid=(NP // BN,),
    in_specs=[_rsp_spec, _mat_spec, _w_spec],
    out_specs=_mat_spec,
    out_shape=jax.ShapeDtypeStruct((NP, D), jnp.float32),
)

_k5_call = pl.pallas_call(
    _k5_body,
    grid=(NP // BN,),
    in_specs=[_rsp_spec, _par_spec, _w_spec],
    out_specs=_mat_spec,
    out_shape=jax.ShapeDtypeStruct((NP, D), jnp.float32),
)

_k6_call = pl.pallas_call(
    _k6_body,
    grid=(NP // BN,),
    in_specs=[_rsp_spec, _par_spec],
    out_specs=_mat_spec,
    out_shape=jax.ShapeDtypeStruct((NP, D), jnp.float32),
)


def kernel(x, edge_index, edge_values, W0, W1):
    row = edge_index[0]
    col = edge_index[1]
    pad = EP - E
    pad_idx = (jnp.arange(pad, dtype=jnp.int32) % N)
    row_p = jnp.concatenate([row, pad_idx])
    col_p = jnp.concatenate([col, pad_idx])
    ev_p = jnp.concatenate([edge_values, jnp.zeros((pad,), jnp.float32)])
    z1 = jnp.zeros((NACC,), jnp.float32)
    z2 = jnp.zeros((NACC2, D), jnp.float32)
    x_p = jnp.concatenate([x, jnp.zeros((NP - N, D), jnp.float32)])

    rsp = _rowsum_call()(row_p, ev_p, z1)
    g1 = _k2_call(rsp, x_p, W0)
    hp = _spmm_call()(g1, row_p, col_p, ev_p, z2)
    g2 = _k5_call(rsp, hp, W1)
    op = _spmm_call()(g2, row_p, col_p, ev_p, z2)
    return _k6_call(rsp, op)[:N]


# speedup vs baseline: 1.0930x; 1.0594x over previous
import functools

import jax
import jax.numpy as jnp
from jax import lax
from jax.experimental import pallas as pl
from jax.experimental.pallas import tpu as pltpu
from jax.experimental.pallas import tpu_sc as plsc

N = 10000
E = 320000
D = 128
NC = 2
NS = 16
NW = NC * NS
EP_TILE = 10240
EP = EP_TILE * NW
KW = 64
NWIN = EP_TILE // KW
NBUF = 5
NP = 10240
NACC = NP
ZCH = NACC // NS
NROW_T = NP // NS
BN = 1024

def _mesh():
    return plsc.VectorSubcoreMesh(
        core_axis_name="c", subcore_axis_name="s",
        num_cores=NC, num_subcores=NS)


KW1 = 128
NWIN1 = EP_TILE // KW1
NB1 = 4


def _rowsum_body(epk_hbm, ev_hbm, z_hbm, out_hbm, *sc):
    ebufs = sc[0:NB1]
    evfs = sc[NB1:2 * NB1]
    acc = sc[2 * NB1]
    sem = sc[2 * NB1 + 1]
    esems = sc[2 * NB1 + 2:2 * NB1 + 2 + NB1]
    ssems = sc[2 * NB1 + 2 + NB1:2 * NB1 + 2 + 2 * NB1]

    c = lax.axis_index("c")
    s = lax.axis_index("s")
    wid = c * NS + s
    base = wid * EP_TILE

    zcp = pltpu.async_copy(z_hbm.at[pl.ds(s * ZCH, ZCH)],
                           acc.at[pl.ds(s * ZCH, ZCH)], sem)

    def estart(b, w):
        pltpu.async_copy(epk_hbm.at[:, pl.ds(base + w * KW1, KW1)],
                         ebufs[b], esems[b])
        pltpu.async_copy(ev_hbm.at[pl.ds(base + w * KW1, KW1)],
                         evfs[b], esems[b])

    def ewait(b, w):
        pltpu.make_async_copy(epk_hbm.at[:, pl.ds(base + w * KW1, KW1)],
                              ebufs[b], esems[b]).wait()
        pltpu.make_async_copy(ev_hbm.at[pl.ds(base + w * KW1, KW1)],
                              evfs[b], esems[b]).wait()

    def sstart(b):
        pltpu.async_copy(evfs[b], acc.at[ebufs[b].at[0]], ssems[b], add=True)

    def swait(b):
        pltpu.make_async_copy(evfs[b], acc.at[ebufs[b].at[0]], ssems[b]).wait()

    def win_ops(w, b):
        @pl.when(w >= 2)
        def _():
            swait((b + 2) % NB1)

        @pl.when(w <= NWIN1 - 3)
        def _():
            estart((b + 2) % NB1, w + 2)

        ewait(b, w)
        sstart(b)

    estart(0, 0)
    estart(1, 1)
    zcp.wait()
    plsc.subcore_barrier()
    lax.fori_loop(
        0, NWIN1 // NB1,
        lambda k, carry: ([win_ops(NB1 * k + j, j) for j in range(NB1)],
                          carry)[1], 0)
    swait((NWIN1 - 2) % NB1)
    swait((NWIN1 - 1) % NB1)
    plsc.subcore_barrier()
    pltpu.sync_copy(acc.at[pl.ds(s * ZCH, ZCH)],
                    out_hbm.at[c, pl.ds(s * ZCH, ZCH)])


@functools.cache
def _rowsum_call():
    return pl.kernel(
        _rowsum_body,
        out_type=jax.ShapeDtypeStruct((NC, NACC), jnp.float32),
        mesh=_mesh(),
        scratch_types=(
            [pltpu.VMEM((2, KW1), jnp.int32) for _ in range(NB1)]
            + [pltpu.VMEM((KW1,), jnp.float32) for _ in range(NB1)]
            + [pltpu.VMEM_SHARED((NACC,), jnp.float32)]
            + [pltpu.SemaphoreType.DMA] * (1 + 2 * NB1)
        ),
    )


def _spmm_body(g_hbm, row_hbm, col_hbm, ev_hbm, z_hbm, out_hbm, *sc):
    rowbufs = sc[0:NBUF]
    colbufs = sc[NBUF:2 * NBUF]
    evbufs = sc[2 * NBUF:3 * NBUF]
    rows = sc[3 * NBUF:4 * NBUF]
    acc = sc[4 * NBUF]
    sem = sc[4 * NBUF + 1]
    esems = sc[4 * NBUF + 2:4 * NBUF + 2 + NBUF]
    gsems = sc[4 * NBUF + 2 + NBUF:4 * NBUF + 2 + 2 * NBUF]
    ssems = sc[4 * NBUF + 2 + 2 * NBUF:4 * NBUF + 2 + 3 * NBUF]

    c = lax.axis_index("c")
    s = lax.axis_index("s")
    wid = c * NS + s
    base = wid * EP_TILE

    zcp = pltpu.async_copy(z_hbm.at[pl.ds(s * NROW_T, NROW_T)],
                           acc.at[pl.ds(s * NROW_T, NROW_T)], sem)

    def estart(b, w):
        o = base + w * KW
        pltpu.async_copy(row_hbm.at[pl.ds(o, KW)], rowbufs[b], esems[b])
        pltpu.async_copy(col_hbm.at[pl.ds(o, KW)], colbufs[b], esems[b])
        pltpu.async_copy(ev_hbm.at[pl.ds(o, KW)], evbufs[b], esems[b])

    def ewait(b, w):
        o = base + w * KW
        pltpu.make_async_copy(
            row_hbm.at[pl.ds(o, KW)], rowbufs[b], esems[b]).wait()
        pltpu.make_async_copy(
            col_hbm.at[pl.ds(o, KW)], colbufs[b], esems[b]).wait()
        pltpu.make_async_copy(
            ev_hbm.at[pl.ds(o, KW)], evbufs[b], esems[b]).wait()

    def gstart(b):
        pltpu.async_copy(g_hbm.at[colbufs[b]], rows[b], gsems[b])

    def gwait(b):
        pltpu.make_async_copy(g_hbm.at[colbufs[b]], rows[b], gsems[b]).wait()

    def scale(b):
        def scale16(e16, carry2):
            e0 = e16 * 16
            ew16 = evbufs[b][pl.ds(e0, 16)]
            for j in range(16):
                bc = jnp.full((16,), ew16[j], jnp.float32)
                for f in range(D // 16):
                    rows[b][e0 + j, pl.ds(f * 16, 16)] = (
                        rows[b][e0 + j, pl.ds(f * 16, 16)] * bc)
            return carry2

        lax.fori_loop(0, KW // 16, scale16, 0)

    def sstart(b):
        pltpu.async_copy(rows[b], acc.at[rowbufs[b]], ssems[b], add=True)

    def swait(b):
        pltpu.make_async_copy(rows[b], acc.at[rowbufs[b]], ssems[b]).wait()

    def win_ops(w, b):
        bp = (b + 3) % NBUF
        bg = (b + 2) % NBUF

        @pl.when(w >= 2)
        def _():
            swait(bp)

        @pl.when(w <= NWIN - 4)
        def _():
            estart(bp, w + 3)

        @pl.when(w <= NWIN - 3)
        def _():
            ewait(bg, w + 2)
            gstart(bg)

        gwait(b)
        scale(b)
        sstart(b)

    estart(0, 0)
    estart(1, 1)
    estart(2, 2)
    ewait(0, 0)
    gstart(0)
    ewait(1, 1)
    gstart(1)
    zcp.wait()
    plsc.subcore_barrier()
    lax.fori_loop(
        0, NWIN // NBUF,
        lambda k, carry: ([win_ops(NBUF * k + j, j) for j in range(NBUF)],
                          carry)[1], 0)
    swait((NWIN - 2) % NBUF)
    swait((NWIN - 1) % NBUF)
    plsc.subcore_barrier()
    pltpu.sync_copy(acc.at[pl.ds(s * NROW_T, NROW_T)],
                    out_hbm.at[c, pl.ds(s * NROW_T, NROW_T)])


@functools.cache
def _spmm_call():
    return pl.kernel(
        _spmm_body,
        out_type=jax.ShapeDtypeStruct((NC, NP, D), jnp.float32),
        mesh=_mesh(),
        scratch_types=(
            [pltpu.VMEM((KW,), jnp.int32) for _ in range(NBUF)]
            + [pltpu.VMEM((KW,), jnp.int32) for _ in range(NBUF)]
            + [pltpu.VMEM((KW,), jnp.float32) for _ in range(NBUF)]
            + [pltpu.VMEM((KW, D), jnp.float32) for _ in range(NBUF)]
            + [pltpu.VMEM_SHARED((NP, D), jnp.float32)]
            + [pltpu.SemaphoreType.DMA] * (1 + 3 * NBUF)
        ),
    )


def _dvec(rsp_ref):
    rs = rsp_ref[0, :] + rsp_ref[1, :] + 1e-6
    return jnp.clip(lax.rsqrt(rs), 0.0, 10.0)


def _k2_body(rsp_ref, x_ref, w0_ref, g1_ref):
    dv = _dvec(rsp_ref)
    xw = jnp.dot(x_ref[...], w0_ref[...], preferred_element_type=jnp.float32)
    g1_ref[...] = dv[:, None] * xw


def _k5_body(rsp_ref, hp_ref, w1_ref, g2_ref):
    dv = _dvec(rsp_ref)
    h = jax.nn.relu(dv[:, None] * (hp_ref[0] + hp_ref[1]))
    hw = jnp.dot(h, w1_ref[...], preferred_element_type=jnp.float32)
    g2_ref[...] = dv[:, None] * hw


def _k6_body(rsp_ref, op_ref, out_ref):
    dv = _dvec(rsp_ref)
    out_ref[...] = dv[:, None] * (op_ref[0] + op_ref[1])


_rsp_spec = pl.BlockSpec((NC, BN), lambda i: (0, i))
_mat_spec = pl.BlockSpec((BN, D), lambda i: (i, 0))
_par_spec = pl.BlockSpec((NC, BN, D), lambda i: (0, i, 0))
_w_spec = pl.BlockSpec((D, D), lambda i: (0, 0))

_k2_call = pl.pallas_call(
    _k2_body,
    grid=(NP // BN,),
    in_specs=[_rsp_spec, _mat_spec, _w_spec],
    out_specs=_mat_spec,
    out_shape=jax.ShapeDtypeStruct((NP, D), jnp.float32),
)

_k5_call = pl.pallas_call(
    _k5_body,
    grid=(NP // BN,),
    in_specs=[_rsp_spec, _par_spec, _w_spec],
    out_specs=_mat_spec,
    out_shape=jax.ShapeDtypeStruct((NP, D), jnp.float32),
)

_k6_call = pl.pallas_call(
    _k6_body,
    grid=(NP // BN,),
    in_specs=[_rsp_spec, _par_spec],
    out_specs=_mat_spec,
    out_shape=jax.ShapeDtypeStruct((NP, D), jnp.float32),
)


def kernel(x, edge_index, edge_values, W0, W1):
    row = edge_index[0]
    col = edge_index[1]
    pad = EP - E
    pad_idx = (jnp.arange(pad, dtype=jnp.int32) % N)
    row_p = jnp.concatenate([row, pad_idx])
    col_p = jnp.concatenate([col, pad_idx])
    ev_p = jnp.concatenate([edge_values, jnp.zeros((pad,), jnp.float32)])
    epk = jnp.stack([row_p, col_p])
    z1 = jnp.zeros((NACC,), jnp.float32)
    z2 = jnp.zeros((NP, D), jnp.float32)
    x_p = jnp.concatenate([x, jnp.zeros((NP - N, D), jnp.float32)])

    rsp = _rowsum_call()(epk, ev_p, z1)
    g1 = _k2_call(rsp, x_p, W0)
    hp = _spmm_call()(g1, row_p, col_p, ev_p, z2)
    g2 = _k5_call(rsp, hp, W1)
    op = _spmm_call()(g2, row_p, col_p, ev_p, z2)
    return _k6_call(rsp, op)[:N]
